# baseline (speedup 1.0000x reference)
import functools

import jax
import jax.numpy as jnp
from jax import lax
from jax.experimental import pallas as pl
from jax.experimental.pallas import tpu as pltpu
from jax.experimental.pallas import tpu_sc as plsc

N = 10000
D = 128

NC = 2
NS = 16
NW = NC * NS
K = 128

NBUF = 4
IBUF = 4
RBUF = 2

CPT0 = 112
CPT1 = 48

NP = 10112
ROWS_PT = NP // NS


def _mesh():
  return plsc.VectorSubcoreMesh(core_axis_name="c", subcore_axis_name="s")


def _make_deg_kernel(cpt):
  @functools.partial(
      pl.kernel,
      mesh=_mesh(),
      out_type=jax.ShapeDtypeStruct((NC, NP, D), jnp.float32),
      scratch_types=[
          pltpu.VMEM_SHARED((NP, D), jnp.float32),
          pltpu.VMEM((cpt, K), jnp.int32),
          pltpu.VMEM((K, D), jnp.float32),
          pltpu.SemaphoreType.DMA((NBUF,)),
      ],
  )
  def deg_kernel(dst_hbm, ones_hbm, zeros_hbm, out_hbm, deg_sh, dst_all, ones_v,
                 dsem):
    c = lax.axis_index("c")
    s = lax.axis_index("s")
    w = s * NC + c
    pltpu.sync_copy(zeros_hbm, deg_sh.at[pl.ds(s * ROWS_PT, ROWS_PT)])
    pltpu.sync_copy(ones_hbm, ones_v)
    pltpu.sync_copy(dst_hbm.at[w], dst_all)
    plsc.subcore_barrier()

    def dwait(j):
      pltpu.make_async_copy(ones_v, deg_sh.at[dst_all.at[j]],
                            dsem.at[lax.rem(j, NBUF)]).wait()

    def step(j, carry):
      @pl.when(j >= NBUF)
      def _():
        dwait(j - NBUF)
      pltpu.async_copy(ones_v, deg_sh.at[dst_all.at[j]],
                       dsem.at[lax.rem(j, NBUF)], add=True)
      return carry

    lax.fori_loop(0, cpt, step, 0)
    for j in range(cpt - NBUF, cpt):
      dwait(j)
    plsc.subcore_barrier()
    pltpu.sync_copy(
        deg_sh.at[pl.ds(s * ROWS_PT, ROWS_PT)],
        out_hbm.at[c, pl.ds(s * ROWS_PT, ROWS_PT)],
    )

  return deg_kernel


def _make_agg_kernel(cpt0, cpt1):
  @functools.partial(
      pl.kernel,
      mesh=_mesh(),
      out_type=jax.ShapeDtypeStruct((NC, NP, D), jnp.float32),
      scratch_types=[
          pltpu.VMEM_SHARED((NP, D), jnp.float32),
          pltpu.VMEM((IBUF, 1, K), jnp.int32),
          pltpu.VMEM((cpt0, K), jnp.int32),
          pltpu.VMEM((RBUF, K, D), jnp.float32),
          pltpu.SemaphoreType.DMA((IBUF,)),
          pltpu.SemaphoreType.DMA((RBUF,)),
      ],
  )
  def agg_kernel(src_hbm, dstc0_hbm, dstc1_hbm, tab_hbm, zeros_hbm, out_hbm,
                 agg_sh, src_ring, dst_all, rows_v, isem, gsem):
    c = lax.axis_index("c")
    s = lax.axis_index("s")
    cnt = jnp.where(c == 0, cpt0, cpt1)
    base = jnp.where(c == 0, s * cpt0, NS * cpt0 + s * cpt1)
    pltpu.sync_copy(zeros_hbm, agg_sh.at[pl.ds(s * ROWS_PT, ROWS_PT)])

    @pl.when(c == 0)
    def _():
      pltpu.sync_copy(dstc0_hbm.at[s], dst_all)

    @pl.when(c == 1)
    def _():
      pltpu.sync_copy(dstc1_hbm.at[s], dst_all.at[pl.ds(0, cpt1)])

    def istart(j):
      b = lax.rem(j, IBUF)
      pltpu.async_copy(src_hbm.at[base + j], src_ring.at[b], isem.at[b])

    def iwait(j):
      b = lax.rem(j, IBUF)
      pltpu.make_async_copy(src_hbm.at[base + j], src_ring.at[b],
                            isem.at[b]).wait()

    def gstart(j):
      b = lax.rem(j, RBUF)
      pltpu.async_copy(tab_hbm.at[src_ring.at[lax.rem(j, IBUF), 0]],
                       rows_v.at[b], gsem.at[b])

    def gwait(j):
      b = lax.rem(j, RBUF)
      pltpu.make_async_copy(tab_hbm.at[src_ring.at[lax.rem(j, IBUF), 0]],
                            rows_v.at[b], gsem.at[b]).wait()

    plsc.subcore_barrier()

    istart(0)
    istart(1)
    istart(2)
    iwait(0)
    gstart(0)

    def step(j, carry):
      @pl.when(j + 1 < cnt)
      def _():
        iwait(j + 1)
        gstart(j + 1)

      @pl.when(j + 3 < cnt)
      def _():
        istart(j + 3)

      gwait(j)
      pltpu.sync_copy(rows_v.at[lax.rem(j, RBUF)], agg_sh.at[dst_all.at[j]],
                      add=True)
      return carry

    lax.fori_loop(0, cnt, step, 0)
    plsc.subcore_barrier()
    pltpu.sync_copy(
        agg_sh.at[pl.ds(s * ROWS_PT, ROWS_PT)],
        out_hbm.at[c, pl.ds(s * ROWS_PT, ROWS_PT)],
    )

  return agg_kernel


def _hs1_body(x_ref, w_ref, d0_ref, d1_ref, o_ref):
  dinv = lax.rsqrt(d0_ref[...] + d1_ref[...] + 1.0)
  h = jnp.dot(x_ref[...], w_ref[...], preferred_element_type=jnp.float32)
  o_ref[...] = dinv * h


def _mid_body(p0_ref, p1_ref, hs_ref, d0_ref, d1_ref, b_ref, w_ref, o_ref):
  dinv = lax.rsqrt(d0_ref[...] + d1_ref[...] + 1.0)
  hs = hs_ref[...]
  z = dinv * (p0_ref[...] + p1_ref[...] + hs) + b_ref[...]
  x2 = jnp.maximum(z, 0.0)
  h2 = jnp.dot(x2, w_ref[...], preferred_element_type=jnp.float32)
  o_ref[...] = dinv * h2


def _out_body(p0_ref, p1_ref, hs_ref, d0_ref, d1_ref, b_ref, o_ref):
  dinv = lax.rsqrt(d0_ref[...] + d1_ref[...] + 1.0)
  z = dinv * (p0_ref[...] + p1_ref[...] + hs_ref[...]) + b_ref[...]
  m = jnp.max(z, axis=1, keepdims=True)
  zs = z - m
  o_ref[...] = zs - jnp.log(jnp.sum(jnp.exp(zs), axis=1, keepdims=True))


def kernel(x, edge_index, W1, b1, W2, b2):
  n = x.shape[0]
  assert n == N
  e = edge_index.shape[1]

  cpt0, cpt1 = CPT0, CPT1
  tot_need = -(-e // K)
  while NS * (cpt0 + cpt1) < tot_need:
    cpt0 += 8
  ch_pad = NS * (cpt0 + cpt1)
  assert ch_pad % NW == 0
  cpt_deg = ch_pad // NW
  e_pad = ch_pad * K
  pad = e_pad - e

  src = edge_index[0].astype(jnp.int32)
  dst = edge_index[1].astype(jnp.int32)
  src = jnp.concatenate([src, jnp.zeros((pad,), jnp.int32)])
  dst = jnp.concatenate([dst, jnp.full((pad,), N, jnp.int32)])
  src3d = src.reshape(ch_pad, 1, K)
  dstc0 = dst[:NS * cpt0 * K].reshape(NS, cpt0, K)
  dstc1 = dst[NS * cpt0 * K:].reshape(NS, cpt1, K)
  dst_deg = dst.reshape(NW, cpt_deg, K)

  onesD = jnp.ones((K, D), jnp.float32)
  zerosD = jnp.zeros((ROWS_PT, D), jnp.float32)

  deg_kernel = _make_deg_kernel(cpt_deg)
  agg_kernel = _make_agg_kernel(cpt0, cpt1)

  degp = deg_kernel(dst_deg, onesD, zerosD)
  d0 = degp[0, :N, 0:1]
  d1 = degp[1, :N, 0:1]

  b1r = b1.reshape(1, D)
  b2r = b2.reshape(1, D)

  hs1 = pl.pallas_call(
      _hs1_body,
      out_shape=jax.ShapeDtypeStruct((N, D), jnp.float32),
  )(x, W1, d0, d1)

  agg1 = agg_kernel(src3d, dstc0, dstc1, hs1, zerosD)

  hs2 = pl.pallas_call(
      _mid_body,
      out_shape=jax.ShapeDtypeStruct((N, D), jnp.float32),
  )(agg1[0, :N], agg1[1, :N], hs1, d0, d1, b1r, W2)

  agg2 = agg_kernel(src3d, dstc0, dstc1, hs2, zerosD)

  out = pl.pallas_call(
      _out_body,
      out_shape=jax.ShapeDtypeStruct((N, D), jnp.float32),
  )(agg2[0, :N], agg2[1, :N], hs2, d0, d1, b2r)

  return out


# speedup vs baseline: 2.5021x; 2.5021x over previous
import functools

import jax
import jax.numpy as jnp
from jax import lax
from jax.experimental import pallas as pl
from jax.experimental.pallas import tpu as pltpu
from jax.experimental.pallas import tpu_sc as plsc

N = 10000
D = 128

NC = 2
NS = 16
NW = NC * NS
K = 128

NBUF = 4
IBUF = 4
RBUF = 2

CPT0 = 112
CPT1 = 48

NP = 10112
ROWS_PT = NP // NS


def _mesh():
  return plsc.VectorSubcoreMesh(core_axis_name="c", subcore_axis_name="s")


def _make_deg_kernel(cpt):
  @functools.partial(
      pl.kernel,
      mesh=_mesh(),
      out_type=jax.ShapeDtypeStruct((NC, NP, D), jnp.float32),
      scratch_types=[
          pltpu.VMEM_SHARED((NP, D), jnp.float32),
          pltpu.VMEM((cpt, K), jnp.int32),
          pltpu.VMEM((K, D), jnp.float32),
          pltpu.SemaphoreType.DMA((NBUF,)),
      ],
  )
  def deg_kernel(dst_hbm, ones_hbm, zeros_hbm, out_hbm, deg_sh, dst_all, ones_v,
                 dsem):
    c = lax.axis_index("c")
    s = lax.axis_index("s")
    w = s * NC + c
    pltpu.sync_copy(zeros_hbm, deg_sh.at[pl.ds(s * ROWS_PT, ROWS_PT)])
    pltpu.sync_copy(ones_hbm, ones_v)
    pltpu.sync_copy(dst_hbm.at[w], dst_all)
    plsc.subcore_barrier()

    def dwait(j):
      pltpu.make_async_copy(ones_v, deg_sh.at[dst_all.at[j]],
                            dsem.at[lax.rem(j, NBUF)]).wait()

    def step(j, carry):
      @pl.when(j >= NBUF)
      def _():
        dwait(j - NBUF)
      pltpu.async_copy(ones_v, deg_sh.at[dst_all.at[j]],
                       dsem.at[lax.rem(j, NBUF)], add=True)
      return carry

    lax.fori_loop(0, cpt, step, 0)
    for j in range(cpt - NBUF, cpt):
      dwait(j)
    plsc.subcore_barrier()
    pltpu.sync_copy(
        deg_sh.at[pl.ds(s * ROWS_PT, ROWS_PT)],
        out_hbm.at[c, pl.ds(s * ROWS_PT, ROWS_PT)],
    )

  return deg_kernel


def _make_agg_kernel(cpt0, cpt1):
  @functools.partial(
      pl.kernel,
      mesh=_mesh(),
      out_type=jax.ShapeDtypeStruct((NC, NP, D), jnp.float32),
      scratch_types=[
          pltpu.VMEM_SHARED((NP, D), jnp.float32),
          pltpu.VMEM((IBUF, 1, K), jnp.int32),
          pltpu.VMEM((cpt0, K), jnp.int32),
          pltpu.VMEM((RBUF, K, D), jnp.float32),
          pltpu.SemaphoreType.DMA((IBUF,)),
          pltpu.SemaphoreType.DMA((RBUF,)),
      ],
  )
  def agg_kernel(src_hbm, dstc0_hbm, dstc1_hbm, tab_hbm, zeros_hbm, out_hbm,
                 agg_sh, src_ring, dst_all, rows_v, isem, gsem):
    c = lax.axis_index("c")
    s = lax.axis_index("s")
    cnt = jnp.where(c == 0, cpt0, cpt1)
    base = jnp.where(c == 0, s * cpt0, NS * cpt0 + s * cpt1)
    pltpu.sync_copy(zeros_hbm, agg_sh.at[pl.ds(s * ROWS_PT, ROWS_PT)])

    @pl.when(c == 0)
    def _():
      pltpu.sync_copy(dstc0_hbm.at[s], dst_all)

    @pl.when(c == 1)
    def _():
      pltpu.sync_copy(dstc1_hbm.at[s], dst_all.at[pl.ds(0, cpt1)])

    def istart(j):
      b = lax.rem(j, IBUF)
      pltpu.async_copy(src_hbm.at[base + j], src_ring.at[b], isem.at[b])

    def iwait(j):
      b = lax.rem(j, IBUF)
      pltpu.make_async_copy(src_hbm.at[base + j], src_ring.at[b],
                            isem.at[b]).wait()

    def gstart(j):
      b = lax.rem(j, RBUF)
      pltpu.async_copy(tab_hbm.at[src_ring.at[lax.rem(j, IBUF), 0]],
                       rows_v.at[b], gsem.at[b])

    def gwait(j):
      b = lax.rem(j, RBUF)
      pltpu.make_async_copy(tab_hbm.at[src_ring.at[lax.rem(j, IBUF), 0]],
                            rows_v.at[b], gsem.at[b]).wait()

    plsc.subcore_barrier()

    istart(0)
    istart(1)
    istart(2)
    iwait(0)
    gstart(0)

    def step(j, carry):
      @pl.when(j + 1 < cnt)
      def _():
        iwait(j + 1)
        gstart(j + 1)

      @pl.when(j + 3 < cnt)
      def _():
        istart(j + 3)

      gwait(j)
      pltpu.sync_copy(rows_v.at[lax.rem(j, RBUF)], agg_sh.at[dst_all.at[j]],
                      add=True)
      return carry

    lax.fori_loop(0, cnt, step, 0)
    plsc.subcore_barrier()
    pltpu.sync_copy(
        agg_sh.at[pl.ds(s * ROWS_PT, ROWS_PT)],
        out_hbm.at[c, pl.ds(s * ROWS_PT, ROWS_PT)],
    )

  return agg_kernel


def _hs1_body(x_ref, w_ref, d0_ref, d1_ref, o_ref):
  dinv = lax.rsqrt(d0_ref[...] + d1_ref[...] + 1.0)
  h = jnp.dot(x_ref[...], w_ref[...], preferred_element_type=jnp.float32)
  o_ref[...] = dinv * h


def _mid_body(p0_ref, p1_ref, hs_ref, d0_ref, d1_ref, b_ref, w_ref, o_ref):
  dinv = lax.rsqrt(d0_ref[...] + d1_ref[...] + 1.0)
  hs = hs_ref[...]
  z = dinv * (p0_ref[...] + p1_ref[...] + hs) + b_ref[...]
  x2 = jnp.maximum(z, 0.0)
  h2 = jnp.dot(x2, w_ref[...], preferred_element_type=jnp.float32)
  o_ref[...] = dinv * h2


def _out_body(p0_ref, p1_ref, hs_ref, d0_ref, d1_ref, b_ref, o_ref):
  dinv = lax.rsqrt(d0_ref[...] + d1_ref[...] + 1.0)
  z = dinv * (p0_ref[...] + p1_ref[...] + hs_ref[...]) + b_ref[...]
  m = jnp.max(z, axis=1, keepdims=True)
  zs = z - m
  o_ref[...] = zs - jnp.log(jnp.sum(jnp.exp(zs), axis=1, keepdims=True))


def kernel(x, edge_index, W1, b1, W2, b2):
  n = x.shape[0]
  assert n == N
  e = edge_index.shape[1]

  cpt0, cpt1 = CPT0, CPT1
  tot_need = -(-e // K)
  while NS * (cpt0 + cpt1) < tot_need:
    cpt0 += 8
  ch_pad = NS * (cpt0 + cpt1)
  assert ch_pad % NW == 0
  cpt_deg = ch_pad // NW
  e_pad = ch_pad * K
  pad = e_pad - e

  src = edge_index[0].astype(jnp.int32)
  dst = edge_index[1].astype(jnp.int32)
  fill = jnp.arange(pad, dtype=jnp.int32)
  src = jnp.concatenate([src, fill % N])
  dst = jnp.concatenate([dst, N + fill % (NP - N)])
  src3d = src.reshape(ch_pad, 1, K)
  dstc0 = dst[:NS * cpt0 * K].reshape(NS, cpt0, K)
  dstc1 = dst[NS * cpt0 * K:].reshape(NS, cpt1, K)
  dst_deg = dst.reshape(NW, cpt_deg, K)

  onesD = jnp.ones((K, D), jnp.float32)
  zerosD = jnp.zeros((ROWS_PT, D), jnp.float32)

  deg_kernel = _make_deg_kernel(cpt_deg)
  agg_kernel = _make_agg_kernel(cpt0, cpt1)

  degp = deg_kernel(dst_deg, onesD, zerosD)
  d0 = degp[0, :N, 0:1]
  d1 = degp[1, :N, 0:1]

  b1r = b1.reshape(1, D)
  b2r = b2.reshape(1, D)

  hs1 = pl.pallas_call(
      _hs1_body,
      out_shape=jax.ShapeDtypeStruct((N, D), jnp.float32),
  )(x, W1, d0, d1)

  agg1 = agg_kernel(src3d, dstc0, dstc1, hs1, zerosD)

  hs2 = pl.pallas_call(
      _mid_body,
      out_shape=jax.ShapeDtypeStruct((N, D), jnp.float32),
  )(agg1[0, :N], agg1[1, :N], hs1, d0, d1, b1r, W2)

  agg2 = agg_kernel(src3d, dstc0, dstc1, hs2, zerosD)

  out = pl.pallas_call(
      _out_body,
      out_shape=jax.ShapeDtypeStruct((N, D), jnp.float32),
  )(agg2[0, :N], agg2[1, :N], hs2, d0, d1, b2r)

  return out


# speedup vs baseline: 2.9532x; 1.1803x over previous
import functools

import jax
import jax.numpy as jnp
from jax import lax
from jax.experimental import pallas as pl
from jax.experimental.pallas import tpu as pltpu
from jax.experimental.pallas import tpu_sc as plsc

N = 10000
D = 128

NC = 2
NS = 16
NW = NC * NS
K = 128

NBUF = 4
IBUF = 4
RBUF = 2

CPT0 = 80
CPT1 = 80

NP = 10112
ROWS_PT = NP // NS


def _mesh():
  return plsc.VectorSubcoreMesh(core_axis_name="c", subcore_axis_name="s")


def _make_deg_kernel(cpt):
  @functools.partial(
      pl.kernel,
      mesh=_mesh(),
      out_type=jax.ShapeDtypeStruct((NC, NP, D), jnp.float32),
      scratch_types=[
          pltpu.VMEM_SHARED((NP, D), jnp.float32),
          pltpu.VMEM((cpt, K), jnp.int32),
          pltpu.VMEM((K, D), jnp.float32),
          pltpu.SemaphoreType.DMA((NBUF,)),
      ],
  )
  def deg_kernel(dst_hbm, ones_hbm, zeros_hbm, out_hbm, deg_sh, dst_all, ones_v,
                 dsem):
    c = lax.axis_index("c")
    s = lax.axis_index("s")
    w = s * NC + c
    pltpu.sync_copy(zeros_hbm, deg_sh.at[pl.ds(s * ROWS_PT, ROWS_PT)])
    pltpu.sync_copy(ones_hbm, ones_v)
    pltpu.sync_copy(dst_hbm.at[w], dst_all)
    plsc.subcore_barrier()

    def dwait(j):
      pltpu.make_async_copy(ones_v, deg_sh.at[dst_all.at[j]],
                            dsem.at[lax.rem(j, NBUF)]).wait()

    def step(j, carry):
      @pl.when(j >= NBUF)
      def _():
        dwait(j - NBUF)
      pltpu.async_copy(ones_v, deg_sh.at[dst_all.at[j]],
                       dsem.at[lax.rem(j, NBUF)], add=True)
      return carry

    lax.fori_loop(0, cpt, step, 0)
    for j in range(cpt - NBUF, cpt):
      dwait(j)
    plsc.subcore_barrier()
    pltpu.sync_copy(
        deg_sh.at[pl.ds(s * ROWS_PT, ROWS_PT)],
        out_hbm.at[c, pl.ds(s * ROWS_PT, ROWS_PT)],
    )

  return deg_kernel


def _make_agg_kernel(cpt0, cpt1):
  @functools.partial(
      pl.kernel,
      mesh=_mesh(),
      out_type=jax.ShapeDtypeStruct((NC, NP, D), jnp.float32),
      scratch_types=[
          pltpu.VMEM_SHARED((NP, D), jnp.float32),
          pltpu.VMEM((IBUF, 1, K), jnp.int32),
          pltpu.VMEM((cpt0, K), jnp.int32),
          pltpu.VMEM((RBUF, K, D), jnp.float32),
          pltpu.SemaphoreType.DMA((IBUF,)),
          pltpu.SemaphoreType.DMA((RBUF,)),
      ],
  )
  def agg_kernel(src_hbm, dstc0_hbm, dstc1_hbm, tab_hbm, zeros_hbm, out_hbm,
                 agg_sh, src_ring, dst_all, rows_v, isem, gsem):
    c = lax.axis_index("c")
    s = lax.axis_index("s")
    cnt = jnp.where(c == 0, cpt0, cpt1)
    base = jnp.where(c == 0, s * cpt0, NS * cpt0 + s * cpt1)
    pltpu.sync_copy(zeros_hbm, agg_sh.at[pl.ds(s * ROWS_PT, ROWS_PT)])

    @pl.when(c == 0)
    def _():
      pltpu.sync_copy(dstc0_hbm.at[s], dst_all)

    @pl.when(c == 1)
    def _():
      pltpu.sync_copy(dstc1_hbm.at[s], dst_all.at[pl.ds(0, cpt1)])

    def istart(j):
      b = lax.rem(j, IBUF)
      pltpu.async_copy(src_hbm.at[base + j], src_ring.at[b], isem.at[b])

    def iwait(j):
      b = lax.rem(j, IBUF)
      pltpu.make_async_copy(src_hbm.at[base + j], src_ring.at[b],
                            isem.at[b]).wait()

    def gstart(j):
      b = lax.rem(j, RBUF)
      pltpu.async_copy(tab_hbm.at[src_ring.at[lax.rem(j, IBUF), 0]],
                       rows_v.at[b], gsem.at[b])

    def gwait(j):
      b = lax.rem(j, RBUF)
      pltpu.make_async_copy(tab_hbm.at[src_ring.at[lax.rem(j, IBUF), 0]],
                            rows_v.at[b], gsem.at[b]).wait()

    plsc.subcore_barrier()

    istart(0)
    istart(1)
    istart(2)
    iwait(0)
    gstart(0)

    def step(j, carry):
      @pl.when(j + 1 < cnt)
      def _():
        iwait(j + 1)
        gstart(j + 1)

      @pl.when(j + 3 < cnt)
      def _():
        istart(j + 3)

      gwait(j)
      pltpu.sync_copy(rows_v.at[lax.rem(j, RBUF)], agg_sh.at[dst_all.at[j]],
                      add=True)
      return carry

    lax.fori_loop(0, cnt, step, 0)
    plsc.subcore_barrier()
    pltpu.sync_copy(
        agg_sh.at[pl.ds(s * ROWS_PT, ROWS_PT)],
        out_hbm.at[c, pl.ds(s * ROWS_PT, ROWS_PT)],
    )

  return agg_kernel


def _hs1_body(x_ref, w_ref, d0_ref, d1_ref, o_ref):
  dinv = lax.rsqrt(d0_ref[...] + d1_ref[...] + 1.0)
  h = jnp.dot(x_ref[...], w_ref[...], preferred_element_type=jnp.float32)
  o_ref[...] = dinv * h


def _mid_body(p0_ref, p1_ref, hs_ref, d0_ref, d1_ref, b_ref, w_ref, o_ref):
  dinv = lax.rsqrt(d0_ref[...] + d1_ref[...] + 1.0)
  hs = hs_ref[...]
  z = dinv * (p0_ref[...] + p1_ref[...] + hs) + b_ref[...]
  x2 = jnp.maximum(z, 0.0)
  h2 = jnp.dot(x2, w_ref[...], preferred_element_type=jnp.float32)
  o_ref[...] = dinv * h2


def _out_body(p0_ref, p1_ref, hs_ref, d0_ref, d1_ref, b_ref, o_ref):
  dinv = lax.rsqrt(d0_ref[...] + d1_ref[...] + 1.0)
  z = dinv * (p0_ref[...] + p1_ref[...] + hs_ref[...]) + b_ref[...]
  m = jnp.max(z, axis=1, keepdims=True)
  zs = z - m
  o_ref[...] = zs - jnp.log(jnp.sum(jnp.exp(zs), axis=1, keepdims=True))


def kernel(x, edge_index, W1, b1, W2, b2):
  n = x.shape[0]
  assert n == N
  e = edge_index.shape[1]

  cpt0, cpt1 = CPT0, CPT1
  tot_need = -(-e // K)
  while NS * (cpt0 + cpt1) < tot_need:
    cpt0 += 8
  ch_pad = NS * (cpt0 + cpt1)
  assert ch_pad % NW == 0
  cpt_deg = ch_pad // NW
  e_pad = ch_pad * K
  pad = e_pad - e

  src = edge_index[0].astype(jnp.int32)
  dst = edge_index[1].astype(jnp.int32)
  fill = jnp.arange(pad, dtype=jnp.int32)
  src = jnp.concatenate([src, fill % N])
  dst = jnp.concatenate([dst, N + fill % (NP - N)])
  src3d = src.reshape(ch_pad, 1, K)
  dstc0 = dst[:NS * cpt0 * K].reshape(NS, cpt0, K)
  dstc1 = dst[NS * cpt0 * K:].reshape(NS, cpt1, K)
  dst_deg = dst.reshape(NW, cpt_deg, K)

  onesD = jnp.ones((K, D), jnp.float32)
  zerosD = jnp.zeros((ROWS_PT, D), jnp.float32)

  deg_kernel = _make_deg_kernel(cpt_deg)
  agg_kernel = _make_agg_kernel(cpt0, cpt1)

  degp = deg_kernel(dst_deg, onesD, zerosD)
  d0 = degp[0, :N, 0:1]
  d1 = degp[1, :N, 0:1]

  b1r = b1.reshape(1, D)
  b2r = b2.reshape(1, D)

  hs1 = pl.pallas_call(
      _hs1_body,
      out_shape=jax.ShapeDtypeStruct((N, D), jnp.float32),
  )(x, W1, d0, d1)

  agg1 = agg_kernel(src3d, dstc0, dstc1, hs1, zerosD)

  hs2 = pl.pallas_call(
      _mid_body,
      out_shape=jax.ShapeDtypeStruct((N, D), jnp.float32),
  )(agg1[0, :N], agg1[1, :N], hs1, d0, d1, b1r, W2)

  agg2 = agg_kernel(src3d, dstc0, dstc1, hs2, zerosD)

  out = pl.pallas_call(
      _out_body,
      out_shape=jax.ShapeDtypeStruct((N, D), jnp.float32),
  )(agg2[0, :N], agg2[1, :N], hs2, d0, d1, b2r)

  return out


# speedup vs baseline: 3.1890x; 1.0799x over previous
import functools

import jax
import jax.numpy as jnp
from jax import lax
from jax.experimental import pallas as pl
from jax.experimental.pallas import tpu as pltpu
from jax.experimental.pallas import tpu_sc as plsc

N = 10000
D = 128

NC = 2
NS = 16
NW = NC * NS
K = 128

NBUF = 4
ISR = 3
IDR = 4
RBUF = 3

CPT0 = 80
CPT1 = 80

NP = 10112
ROWS_PT = NP // NS


def _mesh():
  return plsc.VectorSubcoreMesh(core_axis_name="c", subcore_axis_name="s")


def _make_deg_kernel(cpt):
  @functools.partial(
      pl.kernel,
      mesh=_mesh(),
      out_type=jax.ShapeDtypeStruct((NC, NP, D), jnp.float32),
      scratch_types=[
          pltpu.VMEM_SHARED((NP, D), jnp.float32),
          pltpu.VMEM((cpt, K), jnp.int32),
          pltpu.VMEM((K, D), jnp.float32),
          pltpu.SemaphoreType.DMA((NBUF,)),
      ],
  )
  def deg_kernel(dst_hbm, ones_hbm, zeros_hbm, out_hbm, deg_sh, dst_all, ones_v,
                 dsem):
    c = lax.axis_index("c")
    s = lax.axis_index("s")
    w = s * NC + c
    pltpu.sync_copy(zeros_hbm, deg_sh.at[pl.ds(s * ROWS_PT, ROWS_PT)])
    pltpu.sync_copy(ones_hbm, ones_v)
    pltpu.sync_copy(dst_hbm.at[w], dst_all)
    plsc.subcore_barrier()

    def dwait(j):
      pltpu.make_async_copy(ones_v, deg_sh.at[dst_all.at[j]],
                            dsem.at[lax.rem(j, NBUF)]).wait()

    def step(j, carry):
      @pl.when(j >= NBUF)
      def _():
        dwait(j - NBUF)
      pltpu.async_copy(ones_v, deg_sh.at[dst_all.at[j]],
                       dsem.at[lax.rem(j, NBUF)], add=True)
      return carry

    lax.fori_loop(0, cpt, step, 0)
    for j in range(cpt - NBUF, cpt):
      dwait(j)
    plsc.subcore_barrier()
    pltpu.sync_copy(
        deg_sh.at[pl.ds(s * ROWS_PT, ROWS_PT)],
        out_hbm.at[c, pl.ds(s * ROWS_PT, ROWS_PT)],
    )

  return deg_kernel


def _make_agg_kernel(cpt0, cpt1):
  @functools.partial(
      pl.kernel,
      mesh=_mesh(),
      out_type=jax.ShapeDtypeStruct((NC, NP, D), jnp.float32),
      scratch_types=[
          pltpu.VMEM_SHARED((NP, D), jnp.float32),
          pltpu.VMEM((ISR, 1, K), jnp.int32),
          pltpu.VMEM((IDR, 1, K), jnp.int32),
          pltpu.VMEM((RBUF, K, D), jnp.float32),
          pltpu.SemaphoreType.DMA((ISR,)),
          pltpu.SemaphoreType.DMA((IDR,)),
          pltpu.SemaphoreType.DMA((RBUF,)),
      ],
  )
  def agg_kernel(src_hbm, dst_hbm, tab_hbm, zeros_hbm, out_hbm,
                 agg_sh, src_ring, dst_ring, rows_v, ssem, dsem, gsem):
    c = lax.axis_index("c")
    s = lax.axis_index("s")
    cnt = jnp.where(c == 0, cpt0, cpt1)
    base = jnp.where(c == 0, s * cpt0, NS * cpt0 + s * cpt1)
    pltpu.sync_copy(zeros_hbm, agg_sh.at[pl.ds(s * ROWS_PT, ROWS_PT)])

    def istart(j):
      pltpu.async_copy(src_hbm.at[base + j], src_ring.at[lax.rem(j, ISR)],
                       ssem.at[lax.rem(j, ISR)])
      pltpu.async_copy(dst_hbm.at[base + j], dst_ring.at[lax.rem(j, IDR)],
                       dsem.at[lax.rem(j, IDR)])

    def iwait(j):
      pltpu.make_async_copy(src_hbm.at[base + j],
                            src_ring.at[lax.rem(j, ISR)],
                            ssem.at[lax.rem(j, ISR)]).wait()
      pltpu.make_async_copy(dst_hbm.at[base + j],
                            dst_ring.at[lax.rem(j, IDR)],
                            dsem.at[lax.rem(j, IDR)]).wait()

    def gstart(j):
      b = lax.rem(j, RBUF)
      pltpu.async_copy(tab_hbm.at[src_ring.at[lax.rem(j, ISR), 0]],
                       rows_v.at[b], gsem.at[b])

    def gwait(j):
      b = lax.rem(j, RBUF)
      pltpu.make_async_copy(tab_hbm.at[src_ring.at[lax.rem(j, ISR), 0]],
                            rows_v.at[b], gsem.at[b]).wait()

    plsc.subcore_barrier()

    istart(0)
    istart(1)
    istart(2)
    iwait(0)
    gstart(0)
    iwait(1)
    gstart(1)

    def step(j, carry):
      @pl.when(j + 2 < cnt)
      def _():
        iwait(j + 2)
        gstart(j + 2)

      gwait(j)

      @pl.when(j + 3 < cnt)
      def _():
        istart(j + 3)

      pltpu.sync_copy(rows_v.at[lax.rem(j, RBUF)],
                      agg_sh.at[dst_ring.at[lax.rem(j, IDR), 0]], add=True)
      return carry

    lax.fori_loop(0, cnt, step, 0)
    plsc.subcore_barrier()
    pltpu.sync_copy(
        agg_sh.at[pl.ds(s * ROWS_PT, ROWS_PT)],
        out_hbm.at[c, pl.ds(s * ROWS_PT, ROWS_PT)],
    )

  return agg_kernel


def _hs1_body(x_ref, w_ref, d0_ref, d1_ref, o_ref):
  dinv = lax.rsqrt(d0_ref[...] + d1_ref[...] + 1.0)
  h = jnp.dot(x_ref[...], w_ref[...], preferred_element_type=jnp.float32)
  o_ref[...] = dinv * h


def _mid_body(p0_ref, p1_ref, hs_ref, d0_ref, d1_ref, b_ref, w_ref, o_ref):
  dinv = lax.rsqrt(d0_ref[...] + d1_ref[...] + 1.0)
  hs = hs_ref[...]
  z = dinv * (p0_ref[...] + p1_ref[...] + hs) + b_ref[...]
  x2 = jnp.maximum(z, 0.0)
  h2 = jnp.dot(x2, w_ref[...], preferred_element_type=jnp.float32)
  o_ref[...] = dinv * h2


def _out_body(p0_ref, p1_ref, hs_ref, d0_ref, d1_ref, b_ref, o_ref):
  dinv = lax.rsqrt(d0_ref[...] + d1_ref[...] + 1.0)
  z = dinv * (p0_ref[...] + p1_ref[...] + hs_ref[...]) + b_ref[...]
  m = jnp.max(z, axis=1, keepdims=True)
  zs = z - m
  o_ref[...] = zs - jnp.log(jnp.sum(jnp.exp(zs), axis=1, keepdims=True))


def kernel(x, edge_index, W1, b1, W2, b2):
  n = x.shape[0]
  assert n == N
  e = edge_index.shape[1]

  cpt0, cpt1 = CPT0, CPT1
  tot_need = -(-e // K)
  while NS * (cpt0 + cpt1) < tot_need:
    cpt0 += 8
  ch_pad = NS * (cpt0 + cpt1)
  assert ch_pad % NW == 0
  cpt_deg = ch_pad // NW
  e_pad = ch_pad * K
  pad = e_pad - e

  src = edge_index[0].astype(jnp.int32)
  dst = edge_index[1].astype(jnp.int32)
  fill = jnp.arange(pad, dtype=jnp.int32)
  src = jnp.concatenate([src, fill % N])
  dst = jnp.concatenate([dst, N + fill % (NP - N)])
  src3d = src.reshape(ch_pad, 1, K)
  dst3d = dst.reshape(ch_pad, 1, K)
  dst_deg = dst.reshape(NW, cpt_deg, K)

  onesD = jnp.ones((K, D), jnp.float32)
  zerosD = jnp.zeros((ROWS_PT, D), jnp.float32)

  deg_kernel = _make_deg_kernel(cpt_deg)
  agg_kernel = _make_agg_kernel(cpt0, cpt1)

  degp = deg_kernel(dst_deg, onesD, zerosD)
  d0 = degp[0, :N, 0:1]
  d1 = degp[1, :N, 0:1]

  b1r = b1.reshape(1, D)
  b2r = b2.reshape(1, D)

  hs1 = pl.pallas_call(
      _hs1_body,
      out_shape=jax.ShapeDtypeStruct((N, D), jnp.float32),
  )(x, W1, d0, d1)

  agg1 = agg_kernel(src3d, dst3d, hs1, zerosD)

  hs2 = pl.pallas_call(
      _mid_body,
      out_shape=jax.ShapeDtypeStruct((N, D), jnp.float32),
  )(agg1[0, :N], agg1[1, :N], hs1, d0, d1, b1r, W2)

  agg2 = agg_kernel(src3d, dst3d, hs2, zerosD)

  out = pl.pallas_call(
      _out_body,
      out_shape=jax.ShapeDtypeStruct((N, D), jnp.float32),
  )(agg2[0, :N], agg2[1, :N], hs2, d0, d1, b2r)

  return out
